# baseline (speedup 1.0000x reference)
import jax
import jax.numpy as jnp
from jax import lax
from jax.experimental import pallas as pl
from jax.experimental.pallas import tpu as pltpu
from jax.experimental.pallas import tpu_sc as plsc

_N = 1024 * 26
_D = 1000
_NC = 2
_NS = 16
_NW = _NC * _NS
_RPW = _N // _NW
_BLK = 32
_NBLK = _RPW // _BLK
_NBUF = 2
_L = 16
_G = _BLK // _L


def _sc_body(idx_hbm, out_hbm, idx_v, buf0, buf1, sem0, sem1):
    bufs = (buf0, buf1)
    sems = (sem0, sem1)
    wid = lax.axis_index("s") * _NC + lax.axis_index("c")
    base = wid * _RPW
    pltpu.sync_copy(idx_hbm.at[pl.ds(base, _RPW)], idx_v)

    lanes = lax.iota(jnp.int32, _L)
    zeros16 = jnp.zeros((_L,), jnp.float32)
    ones16 = jnp.ones((_L,), jnp.float32)

    def zinit(i, c):
        for p in range(_NBUF):
            bufs[p][pl.ds(i * _L, _L)] = zeros16
        return c

    lax.fori_loop(0, _BLK * _D // _L, zinit, 0)

    def scatter(buf, b, val16):
        for g in range(_G):
            idxs = idx_v[pl.ds(b * _BLK + g * _L, _L)]
            pos = (g * _L + lanes) * _D + idxs
            plsc.store_scatter(buf, [pos], val16)

    def dst(b):
        return out_hbm.at[pl.ds((base + b * _BLK) * _D, _BLK * _D)]

    def step(t, c):
        for p in range(_NBUF):
            b = t * _NBUF + p

            @pl.when(t > 0)
            def _wait_and_reset():
                pltpu.make_async_copy(bufs[p], dst(b - _NBUF), sems[p]).wait()
                scatter(bufs[p], b - _NBUF, zeros16)

            scatter(bufs[p], b, ones16)
            pltpu.async_copy(bufs[p], dst(b), sems[p])
        return c

    lax.fori_loop(0, _NBLK // _NBUF, step, 0)
    for p in range(_NBUF):
        last = _NBLK - _NBUF + p
        pltpu.make_async_copy(bufs[p], dst(last), sems[p]).wait()


def _one_hot_flat(flat_idx):
    mesh = plsc.VectorSubcoreMesh(core_axis_name="c", subcore_axis_name="s")
    f = pl.kernel(
        _sc_body,
        out_type=jax.ShapeDtypeStruct((_N * _D,), jnp.float32),
        mesh=mesh,
        scratch_types=[
            pltpu.VMEM((_RPW,), jnp.int32),
            pltpu.VMEM((_BLK * _D,), jnp.float32),
            pltpu.VMEM((_BLK * _D,), jnp.float32),
            pltpu.SemaphoreType.DMA,
            pltpu.SemaphoreType.DMA,
        ],
        compiler_params=pltpu.CompilerParams(needs_layout_passes=False),
    )
    return f(flat_idx)


def kernel(x, ones):
    depth = ones.shape[0]
    flat = x.reshape(-1).astype(jnp.int32)
    out = _one_hot_flat(flat)
    return out.reshape(x.shape + (depth,))


# speedup vs baseline: 1.8737x; 1.8737x over previous
import jax
import jax.numpy as jnp
from jax import lax
from jax.experimental import pallas as pl
from jax.experimental.pallas import tpu as pltpu
from jax.experimental.pallas import tpu_sc as plsc

_B = 1024
_S = 26
_D = 1000
_NC = 2
_NS = 16
_NW = _NC * _NS
_IPW = _B // _NW
_NBUF = 2
_L = 16


def _sc_body(idx_hbm, out_hbm, idx_v, buf0, buf1, sem0, sem1):
    bufs = (buf0, buf1)
    sems = (sem0, sem1)
    wid = lax.axis_index("s") * _NC + lax.axis_index("c")
    i0 = wid * _IPW
    pltpu.sync_copy(idx_hbm.at[pl.ds(i0 * _S, _IPW * _S)], idx_v)

    lanes = lax.iota(jnp.int32, _L)
    zeros16 = jnp.zeros((_L,), jnp.float32)
    ones16 = jnp.ones((_L,), jnp.float32)

    def zrow(j, c):
        for p in range(_NBUF):
            for kk in range(0, _D - _L + 1, _L):
                bufs[p][j, pl.ds(kk, _L)] = zeros16
            bufs[p][j, pl.ds(_D - _L, _L)] = zeros16
        return c

    lax.fori_loop(0, _S, zrow, 0)

    def scatter(buf, ri, val16):
        for j_base in (0, _S - _L):
            j_ids = j_base + lanes
            cols = idx_v[pl.ds(ri * _S + j_base, _L)]
            plsc.store_scatter(buf, [j_ids, cols], val16)

    def dst(ri):
        return out_hbm.at[i0 + ri]

    def step(t, c):
        for p in range(_NBUF):
            ri = t * _NBUF + p

            @pl.when(t > 0)
            def _wait_and_reset():
                pltpu.make_async_copy(bufs[p], dst(ri - _NBUF), sems[p]).wait()
                scatter(bufs[p], ri - _NBUF, zeros16)

            scatter(bufs[p], ri, ones16)
            pltpu.async_copy(bufs[p], dst(ri), sems[p])
        return c

    lax.fori_loop(0, _IPW // _NBUF, step, 0)
    for p in range(_NBUF):
        pltpu.make_async_copy(bufs[p], dst(_IPW - _NBUF + p), sems[p]).wait()


def _one_hot(flat_idx):
    mesh = plsc.VectorSubcoreMesh(core_axis_name="c", subcore_axis_name="s")
    f = pl.kernel(
        _sc_body,
        out_type=jax.ShapeDtypeStruct((_B, _S, _D), jnp.float32),
        mesh=mesh,
        scratch_types=[
            pltpu.VMEM((_IPW * _S,), jnp.int32),
            pltpu.VMEM((_S, _D), jnp.float32),
            pltpu.VMEM((_S, _D), jnp.float32),
            pltpu.SemaphoreType.DMA,
            pltpu.SemaphoreType.DMA,
        ],
        compiler_params=pltpu.CompilerParams(needs_layout_passes=False),
    )
    return f(flat_idx)


def kernel(x, ones):
    flat = x.reshape(-1).astype(jnp.int32)
    return _one_hot(flat)


# speedup vs baseline: 5.7830x; 3.0864x over previous
import jax
import jax.numpy as jnp
from jax import lax
from jax.experimental import pallas as pl
from jax.experimental.pallas import tpu as pltpu
from jax.experimental.pallas import tpu_sc as plsc

_B = 1024
_S = 26
_D = 1000
_NC = 2
_NS = 16
_NW = _NC * _NS
_IT = _B // 128
_NU = _S * _IT
_UPW = -(-_NU // _NW)
_L = 16
_GR = 128 // _L


def _sc_body(xt_hbm, out_hbm, idx_v, buf, sem):
    wid = lax.axis_index("s") * _NC + lax.axis_index("c")

    def unit(u):
        uid = wid + _NW * u
        return uid, uid // _IT, lax.rem(uid, _IT)

    for u in range(_UPW):
        uid, j, it = unit(u)

        @pl.when(uid < _NU)
        def _prefetch():
            pltpu.async_copy(
                xt_hbm.at[j, pl.ds(it * 128, 128)],
                idx_v.at[pl.ds(u * 128, 128)],
                sem,
            )

    lanes = lax.iota(jnp.int32, _L)
    zeros16 = jnp.zeros((_L,), jnp.float32)
    ones16 = jnp.ones((_L,), jnp.float32)

    def zrow(r, c):
        for g in range(_GR):
            buf[r, pl.ds(g * _L, _L)] = zeros16
        return c

    lax.fori_loop(0, _D, zrow, 0)

    for u in range(_UPW):
        uid, j, it = unit(u)

        @pl.when(uid < _NU)
        def _drain():
            pltpu.make_async_copy(
                xt_hbm.at[j, pl.ds(it * 128, 128)],
                idx_v.at[pl.ds(u * 128, 128)],
                sem,
            ).wait()

    def scatter(u, val16):
        for g in range(_GR):
            xv = idx_v[pl.ds(u * 128 + g * _L, _L)]
            plsc.store_scatter(buf, [xv, g * _L + lanes], val16)

    for u in range(_UPW):
        uid, j, it = unit(u)

        @pl.when(uid < _NU)
        def _do_unit():
            scatter(u, ones16)
            pltpu.sync_copy(
                buf, out_hbm.at[j, pl.ds(0, _D), pl.ds(it * 128, 128)]
            )
            scatter(u, zeros16)


def _one_hot(xt):
    mesh = plsc.VectorSubcoreMesh(core_axis_name="c", subcore_axis_name="s")
    f = pl.kernel(
        _sc_body,
        out_type=jax.ShapeDtypeStruct((_S, _D, _B), jnp.float32),
        mesh=mesh,
        scratch_types=[
            pltpu.VMEM((_UPW * 128,), jnp.int32),
            pltpu.VMEM((_D, 128), jnp.float32),
            pltpu.SemaphoreType.DMA,
        ],
        compiler_params=pltpu.CompilerParams(needs_layout_passes=False),
    )
    return f(xt)


def kernel(x, ones):
    xt = jnp.transpose(x.astype(jnp.int32))
    out3 = _one_hot(xt)
    return jnp.transpose(out3, (2, 0, 1))
